# baseline (speedup 1.0000x reference)
import jax
import jax.numpy as jnp
from jax import lax
from jax.experimental import pallas as pl
from jax.experimental.pallas import tpu as pltpu
from jax.experimental.pallas import tpu_sc as plsc

VOCAB = 100000
SEQLEN = 2048
EMBED = 768
BATCH = 4
TOKENS = BATCH * SEQLEN

NC = 2
NS = 16
NW = NC * NS
SPW = SEQLEN // NW
CHUNK = 16
NQ = SPW // CHUNK
LANES = 16
VECS = EMBED // LANES
NB = 2


def _emb_body(ids_hbm, tok_hbm, pos_hbm, out_hbm,
              idx_v, pos_v, gat_v, isem, psem, gsem, osem):
    wid = lax.axis_index("s") * NC + lax.axis_index("c")
    sbase = wid * SPW

    for b in range(BATCH):
        pltpu.async_copy(ids_hbm.at[pl.ds(b * SEQLEN + sbase, SPW)],
                         idx_v.at[pl.ds(b * SPW, SPW)], isem)
    for b in range(BATCH):
        pltpu.make_async_copy(ids_hbm.at[pl.ds(b * SEQLEN + sbase, SPW)],
                              idx_v.at[pl.ds(b * SPW, SPW)], isem).wait()

    def gather_desc(qd, s, b):
        return pltpu.make_async_copy(
            tok_hbm.at[idx_v.at[pl.ds(b * SPW + qd * CHUNK, CHUNK)]],
            gat_v.at[s, b], gsem.at[s])

    def pos_desc(qd, s):
        return pltpu.make_async_copy(
            pos_hbm.at[pl.ds(sbase + qd * CHUNK, CHUNK)],
            pos_v.at[s], psem.at[s])

    def out_desc(qd, s, b):
        orow = b * SEQLEN + sbase + qd * CHUNK
        return pltpu.make_async_copy(
            gat_v.at[s, b], out_hbm.at[pl.ds(orow, CHUNK)], osem.at[s])

    pos_desc(0, 0).start()
    for b in range(BATCH):
        gather_desc(0, 0, b).start()

    @pl.loop(0, NQ, step=NB)
    def quad_group(g):
        for i in range(NB):
            qd = g + i
            s = i
            sn = (i + 1) % NB

            @pl.when(qd + 1 < NQ)
            def _():
                @pl.when(qd >= 1)
                def _():
                    for b in range(BATCH):
                        out_desc(qd - 1, sn, b).wait()
                pos_desc(qd + 1, sn).start()
                for b in range(BATCH):
                    gather_desc(qd + 1, sn, b).start()

            pos_desc(qd, s).wait()
            for b in range(BATCH):
                gather_desc(qd, s, b).wait()

            @plsc.parallel_loop(0, CHUNK, unroll=2)
            def add_row(r):
                for j in range(VECS):
                    sl = pl.ds(j * LANES, LANES)
                    pv = pos_v[s, r, sl]
                    for b in range(BATCH):
                        gat_v[s, b, r, sl] = gat_v[s, b, r, sl] + pv

            for b in range(BATCH):
                out_desc(qd, s, b).start()

    for b in range(BATCH):
        out_desc(NQ - 2, 0, b).wait()
        out_desc(NQ - 1, 1, b).wait()


@jax.jit
def _emb_call(ids_flat, token_embeddings, position_embeddings):
    mesh = plsc.VectorSubcoreMesh(core_axis_name="c", subcore_axis_name="s")
    return pl.kernel(
        _emb_body,
        out_type=jax.ShapeDtypeStruct((TOKENS, EMBED), jnp.float32),
        mesh=mesh,
        scratch_types=[
            pltpu.VMEM((BATCH * SPW,), jnp.int32),
            pltpu.VMEM((NB, CHUNK, EMBED), jnp.float32),
            pltpu.VMEM((NB, BATCH, CHUNK, EMBED), jnp.float32),
            pltpu.SemaphoreType.DMA,
            pltpu.SemaphoreType.DMA((NB,)),
            pltpu.SemaphoreType.DMA((NB,)),
            pltpu.SemaphoreType.DMA((NB,)),
        ],
    )(ids_flat, token_embeddings, position_embeddings)


def kernel(input_ids, token_embeddings, position_embeddings):
    ids_flat = input_ids.reshape(-1).astype(jnp.int32)
    out = _emb_call(ids_flat, token_embeddings, position_embeddings)
    return out.reshape(BATCH, SEQLEN, EMBED)


# speedup vs baseline: 2.1040x; 2.1040x over previous
import jax
import jax.numpy as jnp
from jax import lax
from jax.experimental import pallas as pl
from jax.experimental.pallas import tpu as pltpu
from jax.experimental.pallas import tpu_sc as plsc

VOCAB = 100000
SEQLEN = 2048
EMBED = 768
BATCH = 4
TOKENS = BATCH * SEQLEN

NC = 2
NS = 16
NW = NC * NS
SPW = SEQLEN // NW
CHUNK = 8
NQ = SPW // CHUNK
LANES = 16
VECS = EMBED // LANES
NB = 2


def _emb_body(ids_hbm, tok_hbm, pos_hbm, out_hbm,
              idx_v, pos_v, gat_v, isem, psem, gsem, osem):
    wid = lax.axis_index("s") * NC + lax.axis_index("c")
    sbase = wid * SPW

    for b in range(BATCH):
        pltpu.async_copy(ids_hbm.at[pl.ds(b * SEQLEN + sbase, SPW)],
                         idx_v.at[pl.ds(b * SPW, SPW)], isem)
    ppend = pltpu.async_copy(pos_hbm.at[pl.ds(sbase, SPW)], pos_v, psem)
    for b in range(BATCH):
        pltpu.make_async_copy(ids_hbm.at[pl.ds(b * SEQLEN + sbase, SPW)],
                              idx_v.at[pl.ds(b * SPW, SPW)], isem).wait()

    def gather_desc(qd, s, b):
        return pltpu.make_async_copy(
            tok_hbm.at[idx_v.at[pl.ds(b * SPW + qd * CHUNK, CHUNK)]],
            gat_v.at[s, b], gsem.at[s])

    def out_desc(qd, s, b):
        orow = b * SEQLEN + sbase + qd * CHUNK
        return pltpu.make_async_copy(
            gat_v.at[s, b], out_hbm.at[pl.ds(orow, CHUNK)], osem.at[s])

    for b in range(BATCH):
        gather_desc(0, 0, b).start()
    ppend.wait()

    @pl.loop(0, NQ, step=NB)
    def quad_group(g):
        for i in range(NB):
            qd = g + i
            s = i

            sn = (i + 1) % NB

            @pl.when(qd + 1 < NQ)
            def _():
                @pl.when(qd >= 1)
                def _():
                    for b in range(BATCH):
                        out_desc(qd - 1, sn, b).wait()
                for b in range(BATCH):
                    gather_desc(qd + 1, sn, b).start()

            for b in range(BATCH):
                gather_desc(qd, s, b).wait()

            @plsc.parallel_loop(0, CHUNK, unroll=2)
            def add_row(r):
                pr = qd * CHUNK + r
                for j in range(VECS):
                    sl = pl.ds(j * LANES, LANES)
                    pv = pos_v[pr, sl]
                    for b in range(BATCH):
                        plsc.addupdate(gat_v.at[s, b, r, sl], pv)

            for b in range(BATCH):
                out_desc(qd, s, b).start()

    for b in range(BATCH):
        out_desc(NQ - 2, 0, b).wait()
        out_desc(NQ - 1, 1, b).wait()


@jax.jit
def _emb_call(ids_flat, token_embeddings, position_embeddings):
    mesh = plsc.VectorSubcoreMesh(core_axis_name="c", subcore_axis_name="s")
    return pl.kernel(
        _emb_body,
        out_type=jax.ShapeDtypeStruct((TOKENS, EMBED), jnp.float32),
        mesh=mesh,
        scratch_types=[
            pltpu.VMEM((BATCH * SPW,), jnp.int32),
            pltpu.VMEM((SPW, EMBED), jnp.float32),
            pltpu.VMEM((NB, BATCH, CHUNK, EMBED), jnp.float32),
            pltpu.SemaphoreType.DMA,
            pltpu.SemaphoreType.DMA,
            pltpu.SemaphoreType.DMA((NB,)),
            pltpu.SemaphoreType.DMA((NB,)),
        ],
    )(ids_flat, token_embeddings, position_embeddings)


def kernel(input_ids, token_embeddings, position_embeddings):
    ids_flat = input_ids.reshape(-1).astype(jnp.int32)
    out = _emb_call(ids_flat, token_embeddings, position_embeddings)
    return out.reshape(BATCH, SEQLEN, EMBED)
